# baseline (speedup 1.0000x reference)
import functools
import jax
import jax.numpy as jnp
from jax import lax
from jax.experimental import pallas as pl
from jax.experimental.pallas import tpu as pltpu
from jax.experimental.pallas import tpu_sc as plsc

N = 10000
D = 128
E = 320000
NC = 2
NS = 16
NW = NC * NS
EPW = E // NW
CHUNK = 80
NCHUNK = 126
EPT = NCHUNK * CHUNK
GROUPS = CHUNK // 16
RPT = 624
TAIL = N - NS * RPT


def _bcast_lane(v16, lane):
    idx = jnp.full((16, 1), lane, dtype=jnp.int32)
    return lax.gather(
        v16, idx,
        dimension_numbers=lax.GatherDimensionNumbers(
            offset_dims=(), collapsed_slice_dims=(0,), start_index_map=(0,)),
        slice_sizes=(1,),
        mode=lax.GatherScatterMode.PROMISE_IN_BOUNDS)


def _scale(buf, val_sl, q):
    for g in range(GROUPS):
        v16 = val_sl[q, pl.ds(g * 16, 16)]
        for i in range(16):
            e = g * 16 + i
            sc = _bcast_lane(v16, i)
            for d in range(D // 16):
                sl = pl.ds(d * 16, 16)
                buf[e, sl] = buf[e, sl] * sc


@functools.partial(
    pl.kernel,
    out_type=jax.ShapeDtypeStruct((2 * N, D), jnp.float32),
    mesh=plsc.VectorSubcoreMesh(core_axis_name="c", subcore_axis_name="s",
                                num_cores=NC, num_subcores=NS),
    scratch_types=[
        pltpu.VMEM((NCHUNK, CHUNK), jnp.int32),
        pltpu.VMEM((3, CHUNK), jnp.int32),
        pltpu.VMEM((3, CHUNK), jnp.float32),
        pltpu.VMEM((CHUNK, D), jnp.float32),
        pltpu.VMEM((CHUNK, D), jnp.float32),
        pltpu.VMEM((CHUNK, D), jnp.float32),
        pltpu.VMEM_SHARED((N, D), jnp.float32),
        pltpu.SemaphoreType.DMA,
        pltpu.SemaphoreType.DMA,
        pltpu.SemaphoreType.DMA,
        pltpu.SemaphoreType.DMA,
        pltpu.SemaphoreType.DMA,
        pltpu.SemaphoreType.DMA,
        pltpu.SemaphoreType.DMA,
    ],
)
def _sc_spmm(g_hbm, col_hbm, row_hbm, val_hbm, zero_hbm, p_hbm,
             col_v, row_sl, val_sl, buf0, buf1, buf2, acc_sh,
             sg0, sg1, sg2, ss0, ss1, ss2, sem_i):
    bufs = [buf0, buf1, buf2]
    sgs = [sg0, sg1, sg2]
    sss = [ss0, ss1, ss2]

    c_ax = lax.axis_index("c")
    s_ax = lax.axis_index("s")
    wid = s_ax * NC + c_ax
    base = wid * EPT

    pltpu.sync_copy(zero_hbm.at[pl.ds(0, RPT)],
                    acc_sh.at[pl.ds(s_ax * RPT, RPT)])

    @pl.when(s_ax == 0)
    def _():
        pltpu.sync_copy(zero_hbm.at[pl.ds(0, TAIL)],
                        acc_sh.at[pl.ds(NS * RPT, TAIL)])

    pltpu.sync_copy(col_hbm.at[wid], col_v)
    plsc.subcore_barrier()

    def stage_idx(j, p):
        off = pl.ds(base + j * CHUNK, CHUNK)
        pltpu.async_copy(row_hbm.at[off], row_sl.at[p], sem_i)
        pltpu.async_copy(val_hbm.at[off], val_sl.at[p], sem_i)

    def wait_idx(j, p):
        off = pl.ds(base + j * CHUNK, CHUNK)
        pltpu.make_async_copy(row_hbm.at[off], row_sl.at[p], sem_i).wait()
        pltpu.make_async_copy(val_hbm.at[off], val_sl.at[p], sem_i).wait()

    def gather(j, q):
        pltpu.async_copy(g_hbm.at[col_v.at[j]], bufs[q], sgs[q])

    def wait_gather(j, q):
        pltpu.make_async_copy(g_hbm.at[col_v.at[j]], bufs[q], sgs[q]).wait()

    def scatter(q):
        pltpu.async_copy(bufs[q], acc_sh.at[row_sl.at[q]], sss[q], add=True)

    def wait_scatter(q):
        pltpu.make_async_copy(bufs[q], acc_sh.at[row_sl.at[q]],
                              sss[q]).wait()

    stage_idx(0, 0)
    stage_idx(1, 1)
    gather(0, 0)
    gather(1, 1)

    def body(k, carry):
        for q in range(3):
            c = 3 * k + q
            qn = (q + 2) % 3
            wait_gather(c, q)
            wait_idx(c, q)
            _scale(bufs[q], val_sl, q)
            scatter(q)
            if q == 0:
                @pl.when(k >= 1)
                def _():
                    wait_scatter(qn)

                gather(c + 2, qn)
                stage_idx(c + 2, qn)
            else:
                wait_scatter(qn)

                @pl.when(k <= (NCHUNK // 3) - 2)
                def _():
                    gather(c + 2, qn)
                    stage_idx(c + 2, qn)
        return carry

    lax.fori_loop(0, NCHUNK // 3, body, 0)

    wait_scatter(2)

    plsc.subcore_barrier()
    pltpu.sync_copy(acc_sh.at[pl.ds(s_ax * RPT, RPT)],
                    p_hbm.at[pl.ds(c_ax * N + s_ax * RPT, RPT)])

    @pl.when(s_ax == 0)
    def _():
        pltpu.sync_copy(acc_sh.at[pl.ds(NS * RPT, TAIL)],
                        p_hbm.at[pl.ds(c_ax * N + NS * RPT, TAIL)])


_BLK = 2000


def _mm_a_body(h_ref, w_ref, o_ref):
    o_ref[...] = jnp.dot(h_ref[...], w_ref[...],
                         preferred_element_type=jnp.float32)


def _mm_b_body(h_ref, w_ref, b_ref, p0_ref, p1_ref, o_ref):
    o_ref[...] = (jnp.dot(h_ref[...], w_ref[...],
                          preferred_element_type=jnp.float32)
                  + b_ref[...] + p0_ref[...] + p1_ref[...])


def kernel(H, A_indices, A_values, W, b):
    pad = jnp.zeros((NW, EPT - EPW), jnp.int32)
    col = jnp.concatenate(
        [A_indices[1].astype(jnp.int32).reshape(NW, EPW), pad],
        axis=1).reshape(NW, NCHUNK, CHUNK)
    row = jnp.concatenate(
        [A_indices[0].astype(jnp.int32).reshape(NW, EPW), pad],
        axis=1).reshape(NW * EPT)
    val = jnp.concatenate(
        [A_values.reshape(NW, EPW), pad.astype(jnp.float32)],
        axis=1).reshape(NW * EPT)
    w1t = W[:, :D].T
    w2t = W[:, D:].T
    zeros = jnp.zeros((RPT, D), jnp.float32)
    b2 = b.reshape(1, D)

    G = pl.pallas_call(
        _mm_a_body,
        grid=(N // _BLK,),
        in_specs=[
            pl.BlockSpec((_BLK, D), lambda i: (i, 0)),
            pl.BlockSpec((D, D), lambda i: (0, 0)),
        ],
        out_specs=pl.BlockSpec((_BLK, D), lambda i: (i, 0)),
        out_shape=jax.ShapeDtypeStruct((N, D), jnp.float32),
    )(H, w2t)

    P = _sc_spmm(G, col, row, val, zeros)

    Y = pl.pallas_call(
        _mm_b_body,
        grid=(N // _BLK,),
        in_specs=[
            pl.BlockSpec((_BLK, D), lambda i: (i, 0)),
            pl.BlockSpec((D, D), lambda i: (0, 0)),
            pl.BlockSpec((1, D), lambda i: (0, 0)),
            pl.BlockSpec((_BLK, D), lambda i: (i, 0)),
            pl.BlockSpec((_BLK, D), lambda i: (i + N // _BLK, 0)),
        ],
        out_specs=pl.BlockSpec((_BLK, D), lambda i: (i, 0)),
        out_shape=jax.ShapeDtypeStruct((N, D), jnp.float32),
    )(H, w1t, b2, P, P)

    return Y
